# baseline (speedup 1.0000x reference)
import functools
import math

import jax
import jax.numpy as jnp
from jax import lax
from jax.experimental import pallas as pl
from jax.experimental.pallas import tpu as pltpu
from jax.experimental.pallas import tpu_sc as plsc

EMB = 64
SEQ = 50
NTOK = 16384
VOCAB = 1000000
SCALE = math.sqrt(EMB)
LANES = 16
C = 8 * SEQ


def _emb_lookup(tok3, table, *, nc, ns):
    nw = nc * ns
    chunks = tok3.shape[1]
    rows_per_w = chunks * C
    b_per_w = rows_per_w // SEQ
    mesh = plsc.VectorSubcoreMesh(core_axis_name="c", subcore_axis_name="s")

    @functools.partial(
        pl.kernel,
        out_type=jax.ShapeDtypeStruct((NTOK, SEQ, EMB), jnp.float32),
        mesh=mesh,
        scratch_types=[
            pltpu.VMEM((chunks, C), jnp.int32),
            pltpu.VMEM((C, EMB), jnp.float32),
            pltpu.VMEM((C, EMB), jnp.float32),
            pltpu.VMEM((C // SEQ, SEQ, EMB), jnp.float32),
            pltpu.VMEM((C // SEQ, SEQ, EMB), jnp.float32),
            pltpu.SemaphoreType.DMA,
            pltpu.SemaphoreType.DMA,
            pltpu.SemaphoreType.DMA,
        ],
        compiler_params=pltpu.CompilerParams(use_tc_tiling_on_sc=False),
    )
    def run(tok_hbm, tab_hbm, out_hbm, idx_v, buf0, buf1, wb0, wb1,
            sem0, sem1, sem_w):
        wid = lax.axis_index("s") * nc + lax.axis_index("c")
        base_b = wid * b_per_w
        bufs = (buf0, buf1)
        wbufs = (wb0, wb1)
        sems = (sem0, sem1)

        pltpu.sync_copy(tok_hbm.at[wid], idx_v)

        pltpu.async_copy(tab_hbm.at[idx_v.at[0]], buf0, sem0)
        pltpu.async_copy(tab_hbm.at[idx_v.at[1]], buf1, sem1)

        def scale_rows(src_buf, dst_buf):
            def row(r, carry):
                for j in range(C // SEQ):
                    for k in range(EMB // LANES):
                        sl = pl.ds(k * LANES, LANES)
                        dst_buf[j, r, sl] = src_buf[j * SEQ + r, sl] * SCALE
                return carry
            lax.fori_loop(0, SEQ, row, 0)

        def do_chunk(g, b, *, start_next, first):
            buf = bufs[b]
            wbuf = wbufs[b]
            pltpu.make_async_copy(
                tab_hbm.at[pl.ds(0, C)], buf, sems[b]).wait()
            if not first:
                pltpu.make_async_copy(
                    out_hbm.at[pl.ds(0, C // SEQ)], wbuf, sem_w).wait()
            scale_rows(buf, wbuf)
            if start_next:
                pltpu.async_copy(tab_hbm.at[idx_v.at[g + 2]], buf, sems[b])
            pltpu.async_copy(
                wbuf, out_hbm.at[pl.ds(base_b + g * (C // SEQ), C // SEQ)],
                sem_w)

        for b in range(2):
            do_chunk(b, b, start_next=True, first=True)

        def step(g2, carry):
            for b in range(2):
                do_chunk(g2 * 2 + b, b, start_next=True, first=False)
            return carry

        lax.fori_loop(1, chunks // 2 - 1, step, 0)
        for b in range(2):
            do_chunk(chunks - 2 + b, b, start_next=False, first=False)
        for b in range(2):
            pltpu.make_async_copy(
                out_hbm.at[pl.ds(0, C // SEQ)], wbufs[b], sem_w).wait()

    return run(tok3, table)


def kernel(tokens, table):
    info = plsc.get_sparse_core_info()
    nw = info.num_cores * info.num_subcores
    chunks = NTOK * SEQ // (nw * C)
    tok3 = tokens.astype(jnp.int32).reshape(nw, chunks, C)
    return _emb_lookup(tok3, table, nc=info.num_cores, ns=info.num_subcores)


# speedup vs baseline: 1.0016x; 1.0016x over previous
import functools
import math

import jax
import jax.numpy as jnp
from jax import lax
from jax.experimental import pallas as pl
from jax.experimental.pallas import tpu as pltpu
from jax.experimental.pallas import tpu_sc as plsc

EMB = 64
SEQ = 50
NTOK = 16384
VOCAB = 1000000
SCALE = math.sqrt(EMB)
LANES = 16
C = 4 * SEQ
NB = C // SEQ


def _emb_lookup(tok3, table, *, nc, ns):
    nw = nc * ns
    chunks = tok3.shape[1]
    rows_per_w = chunks * C
    b_per_w = rows_per_w // SEQ
    mesh = plsc.VectorSubcoreMesh(core_axis_name="c", subcore_axis_name="s")

    @functools.partial(
        pl.kernel,
        out_type=jax.ShapeDtypeStruct((NTOK, SEQ, EMB), jnp.float32),
        mesh=mesh,
        scratch_types=[
            pltpu.VMEM((chunks, C), jnp.int32),
            pltpu.VMEM((C, EMB), jnp.float32),
            pltpu.VMEM((C, EMB), jnp.float32),
            pltpu.VMEM((C, EMB), jnp.float32),
            pltpu.VMEM((NB, SEQ, EMB), jnp.float32),
            pltpu.VMEM((NB, SEQ, EMB), jnp.float32),
            pltpu.SemaphoreType.DMA,
            pltpu.SemaphoreType.DMA,
            pltpu.SemaphoreType.DMA,
            pltpu.SemaphoreType.DMA,
            pltpu.SemaphoreType.DMA,
        ],
        compiler_params=pltpu.CompilerParams(use_tc_tiling_on_sc=False),
    )
    def run(tok_hbm, tab_hbm, out_hbm, idx_v, gb0, gb1, gb2, wb0, wb1,
            sg0, sg1, sg2, sw0, sw1):
        wid = lax.axis_index("s") * nc + lax.axis_index("c")
        base_b = wid * b_per_w
        gbufs = (gb0, gb1, gb2)
        gsems = (sg0, sg1, sg2)
        wbufs = (wb0, wb1)
        wsems = (sw0, sw1)

        pltpu.sync_copy(tok_hbm.at[wid], idx_v)

        for g in range(3):
            pltpu.async_copy(tab_hbm.at[idx_v.at[g]], gbufs[g], gsems[g])

        def scale_rows(src_buf, dst_buf):
            def row(r, carry):
                for j in range(NB):
                    for k in range(EMB // LANES):
                        sl = pl.ds(k * LANES, LANES)
                        dst_buf[j, r, sl] = src_buf[j * SEQ + r, sl] * SCALE
                return carry
            lax.fori_loop(0, SEQ, row, 0)

        def do_chunk(g, jg, jw, *, first, issue):
            gbuf = gbufs[jg]
            wbuf = wbufs[jw]
            pltpu.make_async_copy(
                tab_hbm.at[pl.ds(0, C)], gbuf, gsems[jg]).wait()
            if not first:
                pltpu.make_async_copy(
                    out_hbm.at[pl.ds(0, NB)], wbuf, wsems[jw]).wait()
            scale_rows(gbuf, wbuf)
            if issue:
                pltpu.async_copy(tab_hbm.at[idx_v.at[g + 3]], gbuf, gsems[jg])
            pltpu.async_copy(
                wbuf, out_hbm.at[pl.ds(base_b + g * NB, NB)], wsems[jw])

        for g in range(6):
            do_chunk(g, g % 3, g % 2, first=(g < 2), issue=True)

        def step(i, carry):
            for j in range(6):
                do_chunk(i * 6 + j, j % 3, j % 2, first=False, issue=True)
            return carry

        nsteps = (chunks - 6 - 8) // 6
        lax.fori_loop(1, 1 + nsteps, step, 0)
        for g in range(6 + nsteps * 6, chunks):
            do_chunk(g, g % 3, g % 2, first=False, issue=(g + 3 < chunks))
        for b in range(2):
            pltpu.make_async_copy(
                out_hbm.at[pl.ds(0, NB)], wbufs[b], wsems[b]).wait()

    return run(tok3, table)


def kernel(tokens, table):
    info = plsc.get_sparse_core_info()
    nw = info.num_cores * info.num_subcores
    chunks = NTOK * SEQ // (nw * C)
    tok3 = tokens.astype(jnp.int32).reshape(nw, chunks, C)
    return _emb_lookup(tok3, table, nc=info.num_cores, ns=info.num_subcores)
